# baseline (device time: 291380 ns/iter reference)
import jax
import jax.numpy as jnp
from jax import lax
from jax.experimental import pallas as pl
from jax.experimental.pallas import tpu as pltpu

N_DEV = 4
M_GLOBAL = 8192
D = 2048
M_CHUNK = M_GLOBAL // N_DEV
TILE = 256
T = M_CHUNK // TILE
HALF = TILE // 2
N_HOP = N_DEV - 1


def kernel(partial, gamma):

    def body(part_ref, gamma_ref, out_ref, commR, commL, stage,
             copy_sems, out_sems, sendR, recvR, sendL, recvL):
        my = lax.axis_index("i")
        left = lax.rem(my + N_DEV - 1, N_DEV)
        right = lax.rem(my + 1, N_DEV)

        barrier_sem = pltpu.get_barrier_semaphore()
        for nbr in (left, right):
            pl.semaphore_signal(
                barrier_sem, inc=1,
                device_id=(nbr,), device_id_type=pl.DeviceIdType.MESH,
            )
        pl.semaphore_wait(barrier_sem, 2)

        def own_rows(d, c, t):
            row0 = c * M_CHUNK + t * TILE + d * HALF
            return part_ref.at[0, pl.ds(row0, HALF), :]

        def make_rdma(d, t, s):
            comm = commR if d == 0 else commL
            if s == 0:
                c = lax.rem(my + N_DEV - 1, N_DEV) if d == 0 \
                    else lax.rem(my + 1, N_DEV)
                src = own_rows(d, c, t)
            else:
                src = comm.at[t, (s - 1) % 2]
            send = (sendR if d == 0 else sendL).at[s * T + t]
            recv = (recvR if d == 0 else recvL).at[s * T + t]
            return pltpu.make_async_remote_copy(
                src_ref=src,
                dst_ref=comm.at[t, s % 2],
                send_sem=send,
                recv_sem=recv,
                device_id=(right,) if d == 0 else (left,),
                device_id_type=pl.DeviceIdType.MESH,
            )

        def make_prefetch(d, t, s):
            c = lax.rem(my + 2 * N_DEV - 2 - s, N_DEV) if d == 0 \
                else lax.rem(my + s + 2, N_DEV)
            return pltpu.make_async_copy(
                own_rows(d, c, t),
                stage.at[d, t],
                copy_sems.at[d * T + t],
            )

        rdmas = {}
        prefetch = {}
        for t in range(T):
            for d in range(2):
                rdmas[d, t] = make_rdma(d, t, 0)
                rdmas[d, t].start()
        for t in range(T):
            for d in range(2):
                prefetch[d, t] = make_prefetch(d, t, 0)
                prefetch[d, t].start()

        out_copies = []
        for s in range(N_HOP):
            for t in range(T):
                for d in range(2):
                    rdmas[d, t].wait()
                    prefetch[d, t].wait()
                comm = (commR, commL)
                if s < N_HOP - 1:
                    for d in range(2):
                        comm[d][t, s % 2] = (
                            comm[d][t, s % 2] + stage[d, t]
                        )
                        rdmas[d, t] = make_rdma(d, t, s + 1)
                        rdmas[d, t].start()
                        prefetch[d, t] = make_prefetch(d, t, s + 1)
                        prefetch[d, t].start()
                else:
                    for d in range(2):
                        y = comm[d][t, (N_HOP - 1) % 2] + stage[d, t]
                        ms = jnp.mean(y * y, axis=-1, keepdims=True)
                        comm[d][t, (N_HOP - 2) % 2] = (
                            y * lax.rsqrt(ms + 1e-6) * gamma_ref[...][None, :]
                        )
                        oc = pltpu.make_async_copy(
                            comm[d].at[t, (N_HOP - 2) % 2],
                            out_ref.at[pl.ds(t * TILE + d * HALF, HALF), :],
                            out_sems.at[d * T + t],
                        )
                        oc.start()
                        out_copies.append(oc)
        for oc in out_copies:
            oc.wait()

    n_sems = N_HOP * T
    return pl.pallas_call(
        body,
        out_shape=jax.ShapeDtypeStruct((M_CHUNK, D), jnp.float32),
        in_specs=[
            pl.BlockSpec(memory_space=pltpu.MemorySpace.HBM),
            pl.BlockSpec(memory_space=pltpu.MemorySpace.VMEM),
        ],
        out_specs=pl.BlockSpec(memory_space=pltpu.MemorySpace.HBM),
        scratch_shapes=[
            pltpu.VMEM((T, 2, HALF, D), jnp.float32),
            pltpu.VMEM((T, 2, HALF, D), jnp.float32),
            pltpu.VMEM((2, T, HALF, D), jnp.float32),
            pltpu.SemaphoreType.DMA((2 * T,)),
            pltpu.SemaphoreType.DMA((2 * T,)),
            pltpu.SemaphoreType.DMA((n_sems,)),
            pltpu.SemaphoreType.DMA((n_sems,)),
            pltpu.SemaphoreType.DMA((n_sems,)),
            pltpu.SemaphoreType.DMA((n_sems,)),
        ],
        compiler_params=pltpu.CompilerParams(
            collective_id=0,
            vmem_limit_bytes=60 * 1024 * 1024,
        ),
    )(partial, gamma)


# device time: 290752 ns/iter; 1.0022x vs baseline; 1.0022x over previous
import jax
import jax.numpy as jnp
from jax import lax
from jax.experimental import pallas as pl
from jax.experimental.pallas import tpu as pltpu

N_DEV = 4
M_GLOBAL = 8192
D = 2048
M_CHUNK = M_GLOBAL // N_DEV
TILE = 512
T = M_CHUNK // TILE
HALF = TILE // 2
N_HOP = N_DEV - 1


def kernel(partial, gamma):

    def body(part_ref, gamma_ref, out_ref, commR, commL, stage,
             copy_sems, out_sems, sendR, recvR, sendL, recvL):
        my = lax.axis_index("i")
        left = lax.rem(my + N_DEV - 1, N_DEV)
        right = lax.rem(my + 1, N_DEV)

        barrier_sem = pltpu.get_barrier_semaphore()
        for nbr in (left, right):
            pl.semaphore_signal(
                barrier_sem, inc=1,
                device_id=(nbr,), device_id_type=pl.DeviceIdType.MESH,
            )
        pl.semaphore_wait(barrier_sem, 2)

        def own_rows(d, c, t):
            row0 = c * M_CHUNK + t * TILE + d * HALF
            return part_ref.at[0, pl.ds(row0, HALF), :]

        def make_rdma(d, t, s):
            comm = commR if d == 0 else commL
            if s == 0:
                c = lax.rem(my + N_DEV - 1, N_DEV) if d == 0 \
                    else lax.rem(my + 1, N_DEV)
                src = own_rows(d, c, t)
            else:
                src = comm.at[t, (s - 1) % 2]
            send = (sendR if d == 0 else sendL).at[s * T + t]
            recv = (recvR if d == 0 else recvL).at[s * T + t]
            return pltpu.make_async_remote_copy(
                src_ref=src,
                dst_ref=comm.at[t, s % 2],
                send_sem=send,
                recv_sem=recv,
                device_id=(right,) if d == 0 else (left,),
                device_id_type=pl.DeviceIdType.MESH,
            )

        def make_prefetch(d, t, s):
            c = lax.rem(my + 2 * N_DEV - 2 - s, N_DEV) if d == 0 \
                else lax.rem(my + s + 2, N_DEV)
            return pltpu.make_async_copy(
                own_rows(d, c, t),
                stage.at[d, t],
                copy_sems.at[d * T + t],
            )

        rdmas = {}
        prefetch = {}
        for t in range(T):
            for d in range(2):
                rdmas[d, t] = make_rdma(d, t, 0)
                rdmas[d, t].start()
        for t in range(T):
            for d in range(2):
                prefetch[d, t] = make_prefetch(d, t, 0)
                prefetch[d, t].start()

        out_copies = []
        for s in range(N_HOP):
            for t in range(T):
                for d in range(2):
                    rdmas[d, t].wait()
                    prefetch[d, t].wait()
                comm = (commR, commL)
                if s < N_HOP - 1:
                    for d in range(2):
                        comm[d][t, s % 2] = (
                            comm[d][t, s % 2] + stage[d, t]
                        )
                        rdmas[d, t] = make_rdma(d, t, s + 1)
                        rdmas[d, t].start()
                        prefetch[d, t] = make_prefetch(d, t, s + 1)
                        prefetch[d, t].start()
                else:
                    for d in range(2):
                        y = comm[d][t, (N_HOP - 1) % 2] + stage[d, t]
                        ms = jnp.mean(y * y, axis=-1, keepdims=True)
                        comm[d][t, (N_HOP - 2) % 2] = (
                            y * lax.rsqrt(ms + 1e-6) * gamma_ref[...][None, :]
                        )
                        oc = pltpu.make_async_copy(
                            comm[d].at[t, (N_HOP - 2) % 2],
                            out_ref.at[pl.ds(t * TILE + d * HALF, HALF), :],
                            out_sems.at[d * T + t],
                        )
                        oc.start()
                        out_copies.append(oc)
        for oc in out_copies:
            oc.wait()

    n_sems = N_HOP * T
    return pl.pallas_call(
        body,
        out_shape=jax.ShapeDtypeStruct((M_CHUNK, D), jnp.float32),
        in_specs=[
            pl.BlockSpec(memory_space=pltpu.MemorySpace.HBM),
            pl.BlockSpec(memory_space=pltpu.MemorySpace.VMEM),
        ],
        out_specs=pl.BlockSpec(memory_space=pltpu.MemorySpace.HBM),
        scratch_shapes=[
            pltpu.VMEM((T, 2, HALF, D), jnp.float32),
            pltpu.VMEM((T, 2, HALF, D), jnp.float32),
            pltpu.VMEM((2, T, HALF, D), jnp.float32),
            pltpu.SemaphoreType.DMA((2 * T,)),
            pltpu.SemaphoreType.DMA((2 * T,)),
            pltpu.SemaphoreType.DMA((n_sems,)),
            pltpu.SemaphoreType.DMA((n_sems,)),
            pltpu.SemaphoreType.DMA((n_sems,)),
            pltpu.SemaphoreType.DMA((n_sems,)),
        ],
        compiler_params=pltpu.CompilerParams(
            collective_id=0,
            vmem_limit_bytes=60 * 1024 * 1024,
        ),
    )(partial, gamma)
